# baseline (speedup 1.0000x reference)
import numpy as np
import jax
import jax.numpy as jnp
from jax.experimental import pallas as pl
from jax.experimental.pallas import tpu as pltpu

_BATCH = 1024
_L = 200
_D = 128
_RATIO = 0.25
_BB = 8


def _pool_matrix():
    m = np.zeros((_L, _L), dtype=np.float32)
    m[0, 0] = 1.0
    m[1, 0] = 1.0
    for t in range(2, _L):
        j = t // 2
        m[t, 2 * j - 1] = 0.5
        m[t, 2 * j] = 0.5
    return m


def _body(mask_ref, mat_ref, x_ref, o_ref):
    pmat = mat_ref[...]
    for i in range(_BB):
        x = x_ref[i]
        m = mask_ref[i]
        p = jax.lax.dot(pmat, x, preferred_element_type=jnp.float32)
        o_ref[i] = x + m * (p - x)


def kernel(feats, max_len):
    batch = feats.shape[0]
    num_to_pool = int(batch * _RATIO)
    perm = jax.random.permutation(jax.random.key(1), batch)
    ind = jnp.sort(perm[:num_to_pool])
    mask = jnp.zeros((batch,), jnp.float32).at[ind].set(1.0)
    mask3 = jnp.broadcast_to(mask[:, None, None], (batch, 1, _D))
    pmat = jnp.asarray(_pool_matrix())

    return pl.pallas_call(
        _body,
        grid=(batch // _BB,),
        in_specs=[
            pl.BlockSpec((_BB, 1, _D), lambda i: (i, 0, 0)),
            pl.BlockSpec((_L, _L), lambda i: (0, 0)),
            pl.BlockSpec((_BB, _L, _D), lambda i: (i, 0, 0)),
        ],
        out_specs=pl.BlockSpec((_BB, _L, _D), lambda i: (i, 0, 0)),
        out_shape=jax.ShapeDtypeStruct(feats.shape, feats.dtype),
        compiler_params=pltpu.CompilerParams(
            dimension_semantics=("parallel",),
        ),
    )(mask3, pmat, feats)


# speedup vs baseline: 1.7374x; 1.7374x over previous
import jax
import jax.numpy as jnp
from jax.experimental import pallas as pl
from jax.experimental.pallas import tpu as pltpu

_L = 200
_D = 128
_RATIO = 0.25
_BB = 32


def _body(flags_ref, x_ref, o_ref):
    i = pl.program_id(0)
    for j in range(_BB):
        flag = flags_ref[i * _BB + j]

        @pl.when(flag == 0)
        def _copy():
            o_ref[j] = x_ref[j]

        @pl.when(flag != 0)
        def _pool():
            x = x_ref[j]
            xm1 = jnp.concatenate([x[:1], x[:-1]], axis=0)
            avg = 0.5 * (x + xm1)
            avg_sh = jnp.concatenate([avg[:1], avg[:-1]], axis=0)
            t = jax.lax.broadcasted_iota(jnp.int32, (_L, _D), 0)
            o_ref[j] = jnp.where((t % 2) == 0, avg, avg_sh)


def kernel(feats, max_len):
    batch = feats.shape[0]
    num_to_pool = int(batch * _RATIO)
    perm = jax.random.permutation(jax.random.key(1), batch)
    ind = jnp.sort(perm[:num_to_pool])
    flags = jnp.zeros((batch,), jnp.int32).at[ind].set(1)

    grid_spec = pltpu.PrefetchScalarGridSpec(
        num_scalar_prefetch=1,
        grid=(batch // _BB,),
        in_specs=[pl.BlockSpec((_BB, _L, _D), lambda i, flags: (i, 0, 0))],
        out_specs=pl.BlockSpec((_BB, _L, _D), lambda i, flags: (i, 0, 0)),
    )
    return pl.pallas_call(
        _body,
        grid_spec=grid_spec,
        out_shape=jax.ShapeDtypeStruct(feats.shape, feats.dtype),
        compiler_params=pltpu.CompilerParams(
            dimension_semantics=("parallel",),
        ),
    )(flags, feats)


# speedup vs baseline: 1.7529x; 1.0089x over previous
import jax
import jax.numpy as jnp
from jax.experimental import pallas as pl
from jax.experimental.pallas import tpu as pltpu

_L = 200
_D = 128
_RATIO = 0.25
_BB = 32


def _body(flags_ref, x_ref, o_ref):
    i = pl.program_id(0)
    o_ref[...] = x_ref[...]
    even = (jax.lax.broadcasted_iota(jnp.int32, (_L, _D), 0) % 2) == 0
    for j in range(_BB):
        flag = flags_ref[i * _BB + j]

        @pl.when(flag != 0)
        def _pool():
            x = x_ref[j]
            xm1 = jnp.concatenate([x[:1], x[:-1]], axis=0)
            avg = 0.5 * (x + xm1)
            avg_sh = jnp.concatenate([avg[:1], avg[:-1]], axis=0)
            o_ref[j] = jnp.where(even, avg, avg_sh)


def kernel(feats, max_len):
    batch = feats.shape[0]
    num_to_pool = int(batch * _RATIO)
    perm = jax.random.permutation(jax.random.key(1), batch)
    ind = jnp.sort(perm[:num_to_pool])
    flags = jnp.zeros((batch,), jnp.int32).at[ind].set(1)

    grid_spec = pltpu.PrefetchScalarGridSpec(
        num_scalar_prefetch=1,
        grid=(batch // _BB,),
        in_specs=[pl.BlockSpec((_BB, _L, _D), lambda i, flags: (i, 0, 0))],
        out_specs=pl.BlockSpec((_BB, _L, _D), lambda i, flags: (i, 0, 0)),
    )
    return pl.pallas_call(
        _body,
        grid_spec=grid_spec,
        out_shape=jax.ShapeDtypeStruct(feats.shape, feats.dtype),
        compiler_params=pltpu.CompilerParams(
            dimension_semantics=("parallel",),
        ),
    )(flags, feats)


# speedup vs baseline: 1.8115x; 1.0334x over previous
import jax
import jax.numpy as jnp
from jax.experimental import pallas as pl
from jax.experimental.pallas import tpu as pltpu

_L = 200
_D = 128
_RATIO = 0.25
_BB = 64


def _body(flags_ref, x_ref, o_ref):
    i = pl.program_id(0)
    o_ref[...] = x_ref[...]
    even = (jax.lax.broadcasted_iota(jnp.int32, (_L, _D), 0) % 2) == 0
    for j in range(_BB):
        flag = flags_ref[i * _BB + j]

        @pl.when(flag != 0)
        def _pool():
            x = x_ref[j]
            xm1 = jnp.concatenate([x[:1], x[:-1]], axis=0)
            avg = 0.5 * (x + xm1)
            avg_sh = jnp.concatenate([avg[:1], avg[:-1]], axis=0)
            o_ref[j] = jnp.where(even, avg, avg_sh)


def kernel(feats, max_len):
    batch = feats.shape[0]
    num_to_pool = int(batch * _RATIO)
    perm = jax.random.permutation(jax.random.key(1), batch)
    ind = jnp.sort(perm[:num_to_pool])
    flags = jnp.zeros((batch,), jnp.int32).at[ind].set(1)

    grid_spec = pltpu.PrefetchScalarGridSpec(
        num_scalar_prefetch=1,
        grid=(batch // _BB,),
        in_specs=[pl.BlockSpec((_BB, _L, _D), lambda i, flags: (i, 0, 0))],
        out_specs=pl.BlockSpec((_BB, _L, _D), lambda i, flags: (i, 0, 0)),
    )
    return pl.pallas_call(
        _body,
        grid_spec=grid_spec,
        out_shape=jax.ShapeDtypeStruct(feats.shape, feats.dtype),
        compiler_params=pltpu.CompilerParams(
            dimension_semantics=("parallel",),
        ),
    )(flags, feats)


# speedup vs baseline: 1.8371x; 1.0141x over previous
import jax
import jax.numpy as jnp
from jax.experimental import pallas as pl
from jax.experimental.pallas import tpu as pltpu

_L = 200
_D = 128
_RATIO = 0.25
_BB = 128


def _body(flags_ref, x_ref, o_ref):
    i = pl.program_id(0)
    o_ref[...] = x_ref[...]
    even = (jax.lax.broadcasted_iota(jnp.int32, (_L, _D), 0) % 2) == 0
    for j in range(_BB):
        flag = flags_ref[i * _BB + j]

        @pl.when(flag != 0)
        def _pool():
            x = x_ref[j]
            xm1 = jnp.concatenate([x[:1], x[:-1]], axis=0)
            avg = 0.5 * (x + xm1)
            avg_sh = jnp.concatenate([avg[:1], avg[:-1]], axis=0)
            o_ref[j] = jnp.where(even, avg, avg_sh)


def kernel(feats, max_len):
    batch = feats.shape[0]
    num_to_pool = int(batch * _RATIO)
    perm = jax.random.permutation(jax.random.key(1), batch)
    ind = jnp.sort(perm[:num_to_pool])
    flags = jnp.zeros((batch,), jnp.int32).at[ind].set(1)

    grid_spec = pltpu.PrefetchScalarGridSpec(
        num_scalar_prefetch=1,
        grid=(batch // _BB,),
        in_specs=[pl.BlockSpec((_BB, _L, _D), lambda i, flags: (i, 0, 0))],
        out_specs=pl.BlockSpec((_BB, _L, _D), lambda i, flags: (i, 0, 0)),
    )
    return pl.pallas_call(
        _body,
        grid_spec=grid_spec,
        out_shape=jax.ShapeDtypeStruct(feats.shape, feats.dtype),
        compiler_params=pltpu.CompilerParams(
            dimension_semantics=("parallel",),
        ),
    )(flags, feats)


# speedup vs baseline: 2.2539x; 1.2269x over previous
import functools

import numpy as np
import jax
import jax.numpy as jnp
from jax.experimental import pallas as pl
from jax.experimental.pallas import tpu as pltpu

_L = 200
_D = 128
_RATIO = 0.25
_BB = 128


def _block_lists(batch):
    num = int(batch * _RATIO)
    with jax.ensure_compile_time_eval():
        perm = np.asarray(jax.random.permutation(jax.random.key(1), batch))
    ind = np.sort(perm[:num])
    nblocks = batch // _BB
    lists = []
    for b in range(nblocks):
        local = [int(r - b * _BB) for r in ind if b * _BB <= r < (b + 1) * _BB]
        lists.append(local)
    maxp = max(len(l) for l in lists)
    padded = np.array(
        [l + [l[0]] * (maxp - len(l)) for l in lists], dtype=np.int32
    )
    return padded, maxp


def _body(plist_ref, x_ref, o_ref, *, maxp):
    i = pl.program_id(0)
    o_ref[...] = x_ref[...]
    even = (jax.lax.broadcasted_iota(jnp.int32, (_L, _D), 0) % 2) == 0
    for k in range(maxp):
        j = plist_ref[i * maxp + k]
        x = x_ref[j]
        xm1 = jnp.concatenate([x[:1], x[:-1]], axis=0)
        avg = 0.5 * (x + xm1)
        avg_sh = jnp.concatenate([avg[:1], avg[:-1]], axis=0)
        o_ref[j] = jnp.where(even, avg, avg_sh)


def kernel(feats, max_len):
    batch = feats.shape[0]
    padded, maxp = _block_lists(batch)
    plist = jnp.asarray(padded.reshape(-1))

    grid_spec = pltpu.PrefetchScalarGridSpec(
        num_scalar_prefetch=1,
        grid=(batch // _BB,),
        in_specs=[pl.BlockSpec((_BB, _L, _D), lambda i, plist: (i, 0, 0))],
        out_specs=pl.BlockSpec((_BB, _L, _D), lambda i, plist: (i, 0, 0)),
    )
    return pl.pallas_call(
        functools.partial(_body, maxp=maxp),
        grid_spec=grid_spec,
        out_shape=jax.ShapeDtypeStruct(feats.shape, feats.dtype),
        compiler_params=pltpu.CompilerParams(
            dimension_semantics=("parallel",),
        ),
    )(plist, feats)
